# Initial kernel scaffold; baseline (speedup 1.0000x reference)
import jax
import jax.numpy as jnp
from jax.experimental import pallas as pl


def kernel(x, mask_nonzero, w, u):
    raise NotImplementedError("write your pallas kernel here")


# speedup vs baseline: 2.9001x; 2.9001x over previous
import jax
import jax.numpy as jnp
from jax.experimental import pallas as pl
from jax.experimental.pallas import tpu as pltpu

_BN = 1024
_CHUNK = 2048


def _fused_body(x_ref, bi_ref, ri_ref, roots_ref, w_ref, u_ref,
                out_ref, g_ref, occ_ref, mvec_ref):
    b = pl.program_id(0)
    j = pl.program_id(1)
    L = bi_ref.shape[0]

    @pl.when((b == 0) & (j == 0))
    def _build_occupancy():
        mvec_ref[...] = jnp.zeros_like(mvec_ref)
        occ_ref[...] = jnp.zeros_like(occ_ref)
        lane = jax.lax.broadcasted_iota(jnp.int32, (1, 256), 1)

        def body(c, carry):
            bc = bi_ref[pl.ds(c * _CHUNK, _CHUNK), :]
            rc = ri_ref[pl.ds(c * _CHUNK, _CHUNK), :]
            ids = bc * 16 + rc
            hit = (ids == lane).astype(jnp.float32)
            occ_ref[0:1, :] = jnp.maximum(
                occ_ref[0:1, :], jnp.max(hit, axis=0, keepdims=True))
            return carry

        jax.lax.fori_loop(0, L // _CHUNK, body, 0)

    x_blk = x_ref[0]
    xw = jnp.dot(x_blk, w_ref[...], preferred_element_type=jnp.float32)

    @pl.when(j == 0)
    def _first_block():
        sub = jax.lax.broadcasted_iota(jnp.int32, (16, 256), 0)
        lane = jax.lax.broadcasted_iota(jnp.int32, (16, 256), 1)
        sel = (lane == sub + b * 16).astype(jnp.float32)
        m16 = jnp.sum(sel * occ_ref[0:1, :], axis=1, keepdims=True)
        mvec_ref[0:16, :] = m16
        mvec = mvec_ref[...]
        ohb = (jax.lax.broadcasted_iota(jnp.int32, (1, 16), 1) == b
               ).astype(jnp.float32)
        rv = jnp.dot(ohb, roots_ref[...], preferred_element_type=jnp.float32)
        a_b = jnp.sum(jnp.dot(rv, u_ref[...],
                              preferred_element_type=jnp.float32))
        g = jax.nn.sigmoid(xw + mvec * a_b)
        out_ref[0] = x_blk * g + (mvec * (1.0 - g)) * rv
        g_ref[0] = g

    @pl.when(j != 0)
    def _rest():
        g = jax.nn.sigmoid(xw)
        out_ref[0] = x_blk * g
        g_ref[0] = g


def kernel(x, mask_nonzero, w, u):
    B, N, H = x.shape
    L = mask_nonzero.shape[1]
    NJ = N // _BN
    bi = mask_nonzero[0].reshape(L, 1)
    ri = mask_nonzero[1].reshape(L, 1)
    roots = x[:, 0, :]
    out, g = pl.pallas_call(
        _fused_body,
        grid=(B, NJ),
        in_specs=[
            pl.BlockSpec((1, _BN, H), lambda b, j: (b, j, 0)),
            pl.BlockSpec((L, 1), lambda b, j: (0, 0)),
            pl.BlockSpec((L, 1), lambda b, j: (0, 0)),
            pl.BlockSpec((B, H), lambda b, j: (0, 0)),
            pl.BlockSpec((H, 1), lambda b, j: (0, 0)),
            pl.BlockSpec((H, 1), lambda b, j: (0, 0)),
        ],
        out_specs=[
            pl.BlockSpec((1, _BN, H), lambda b, j: (b, j, 0)),
            pl.BlockSpec((1, _BN, 1), lambda b, j: (b, j, 0)),
        ],
        out_shape=[
            jax.ShapeDtypeStruct((B, N, H), x.dtype),
            jax.ShapeDtypeStruct((B, N, 1), x.dtype),
        ],
        scratch_shapes=[
            pltpu.VMEM((8, 256), jnp.float32),
            pltpu.VMEM((_BN, 1), jnp.float32),
        ],
    )(x, bi, ri, roots, w, u)
    return out, g


# speedup vs baseline: 4.0325x; 1.3905x over previous
import jax
import jax.numpy as jnp
from jax.experimental import pallas as pl
from jax.experimental.pallas import tpu as pltpu

_CHUNK = 2048


def _fused_body(x_ref, bi_ref, ri_ref, roots_ref, w_ref, u_ref,
                out_ref, g_ref, occ_ref):
    b = pl.program_id(0)
    L = bi_ref.shape[0]

    @pl.when(b == 0)
    def _build_occupancy():
        occ_ref[...] = jnp.zeros_like(occ_ref)
        lane = jax.lax.broadcasted_iota(jnp.int32, (1, 256), 1)

        def body(c, carry):
            bc = bi_ref[pl.ds(c * _CHUNK, _CHUNK), :]
            rc = ri_ref[pl.ds(c * _CHUNK, _CHUNK), :]
            ids = bc * 16 + rc
            hit = (ids == lane).astype(jnp.float32)
            occ_ref[0:1, :] = jnp.maximum(
                occ_ref[0:1, :], jnp.max(hit, axis=0, keepdims=True))
            return carry

        jax.lax.fori_loop(0, L // _CHUNK, body, 0)

    x_blk = x_ref[0, 16:, :]
    xw = jnp.dot(x_blk, w_ref[...], preferred_element_type=jnp.float32)
    g = jax.nn.sigmoid(xw)
    out_ref[0, 16:, :] = x_blk * g
    g_ref[0, 16:, :] = g

    sub = jax.lax.broadcasted_iota(jnp.int32, (16, 256), 0)
    lane = jax.lax.broadcasted_iota(jnp.int32, (16, 256), 1)
    sel = (lane == sub + b * 16).astype(jnp.float32)
    m16 = jnp.sum(sel * occ_ref[0:1, :], axis=1, keepdims=True)
    ohb = (jax.lax.broadcasted_iota(jnp.int32, (1, 16), 1) == b
           ).astype(jnp.float32)
    rv = jnp.dot(ohb, roots_ref[...], preferred_element_type=jnp.float32)
    a_b = jnp.sum(jnp.dot(rv, u_ref[...], preferred_element_type=jnp.float32))
    x16 = x_ref[0, 0:16, :]
    xw16 = jnp.dot(x16, w_ref[...], preferred_element_type=jnp.float32)
    g16 = jax.nn.sigmoid(xw16 + m16 * a_b)
    out_ref[0, 0:16, :] = x16 * g16 + (m16 * (1.0 - g16)) * rv
    g_ref[0, 0:16, :] = g16


def kernel(x, mask_nonzero, w, u):
    B, N, H = x.shape
    L = mask_nonzero.shape[1]
    bi = mask_nonzero[0].reshape(L, 1)
    ri = mask_nonzero[1].reshape(L, 1)
    roots = x[:, 0, :]
    out, g = pl.pallas_call(
        _fused_body,
        grid=(B,),
        in_specs=[
            pl.BlockSpec((1, N, H), lambda b: (b, 0, 0)),
            pl.BlockSpec((L, 1), lambda b: (0, 0)),
            pl.BlockSpec((L, 1), lambda b: (0, 0)),
            pl.BlockSpec((B, H), lambda b: (0, 0)),
            pl.BlockSpec((H, 1), lambda b: (0, 0)),
            pl.BlockSpec((H, 1), lambda b: (0, 0)),
        ],
        out_specs=[
            pl.BlockSpec((1, N, H), lambda b: (b, 0, 0)),
            pl.BlockSpec((1, N, 1), lambda b: (b, 0, 0)),
        ],
        out_shape=[
            jax.ShapeDtypeStruct((B, N, H), x.dtype),
            jax.ShapeDtypeStruct((B, N, 1), x.dtype),
        ],
        scratch_shapes=[
            pltpu.VMEM((8, 256), jnp.float32),
        ],
    )(x, bi, ri, roots, w, u)
    return out, g
